# Initial kernel scaffold; baseline (speedup 1.0000x reference)
import jax
import jax.numpy as jnp
from jax.experimental import pallas as pl


def kernel(x, edge_index, atom_ens, batch, mol_feats, W_en, b_en, gc0_W, gc0_b, gc1_W, gc1_b, gc2_W, gc2_b, bn_gc_g, bn_gc_b, fcm0_W, fcm0_b, fcm1_W, fcm1_b, bn_m_g, bn_m_b, fc0_W, fc0_b, fc1_W, fc1_b, fc2_W, fc2_b):
    raise NotImplementedError("write your pallas kernel here")


# speedup vs baseline: 11.2403x; 11.2403x over previous
import functools

import jax
import jax.numpy as jnp
from jax import lax
from jax.experimental import pallas as pl
from jax.experimental.pallas import tpu as pltpu
from jax.experimental.pallas import tpu_sc as plsc

N = 10000
E = 320000
B = 512
D = 128
DH = D // 2

NC, NS, L = 2, 16, 16
TILES = NC * NS
ROWS = 2560
E_PAD = ROWS * 128
R = ROWS // TILES
RPT = ROWS // NS
N_PAD = 10240
STRIPE = N_PAD // NS


def _deg_body(dst_hbm, out_hbm, didx_v, ones_v, zstripe_v, deg_sh):
    c = lax.axis_index("c")
    s = lax.axis_index("s")
    g = s * NC + c
    base = s * STRIPE

    zeros16 = jnp.zeros((L,), jnp.float32)
    ones16 = jnp.full((L,), 1.0, jnp.float32)

    def _fill(i, _):
        zstripe_v[pl.ds(i * L, L)] = zeros16
        return 0

    lax.fori_loop(0, STRIPE // L, _fill, 0)

    def _fill1(i, _):
        ones_v[pl.ds(i * L, L)] = ones16
        return 0

    lax.fori_loop(0, 128 // L, _fill1, 0)

    pltpu.sync_copy(zstripe_v, deg_sh.at[pl.ds(base, STRIPE)])
    pltpu.sync_copy(dst_hbm.at[pl.ds(g * R, R)], didx_v)
    plsc.subcore_barrier()

    def _row(j, _):
        pltpu.sync_copy(ones_v, deg_sh.at[didx_v.at[j]], add=True)
        return 0

    lax.fori_loop(0, R, _row, 0)

    plsc.subcore_barrier()
    pltpu.sync_copy(deg_sh.at[pl.ds(base, STRIPE)], out_hbm.at[c, pl.ds(base, STRIPE)])


def _scatter_body(m2_hbm, src_hbm, dst_hbm, out_hbm,
                  sidx_v, didx_v, rows_v, acc_sh, sem0, sem1):
    c = lax.axis_index("c")
    s = lax.axis_index("s")
    base = s * STRIPE

    pltpu.sync_copy(m2_hbm.at[c, pl.ds(base, STRIPE)], acc_sh.at[pl.ds(base, STRIPE)])
    pltpu.sync_copy(src_hbm.at[pl.ds(s * RPT, RPT)], sidx_v)
    pltpu.sync_copy(dst_hbm.at[pl.ds(s * RPT, RPT)], didx_v)
    plsc.subcore_barrier()

    tab = m2_hbm.at[c]
    sems = (sem0, sem1)

    def _gather_start(j, b):
        pltpu.async_copy(tab.at[sidx_v.at[j]], rows_v.at[b], sems[b])

    def _gather_wait(j, b):
        pltpu.make_async_copy(tab.at[sidx_v.at[j]], rows_v.at[b], sems[b]).wait()

    def _scatter(j, b):
        pltpu.sync_copy(rows_v.at[b], acc_sh.at[didx_v.at[j]], add=True)

    _gather_start(0, 0)

    def _body(jj, _):
        j0 = 2 * jj
        _gather_start(j0 + 1, 1)
        _gather_wait(j0, 0)
        _scatter(j0, 0)

        @pl.when(j0 + 2 < RPT)
        def _():
            _gather_start(j0 + 2, 0)

        _gather_wait(j0 + 1, 1)
        _scatter(j0 + 1, 1)
        return 0

    lax.fori_loop(0, RPT // 2, _body, 0)

    plsc.subcore_barrier()
    pltpu.sync_copy(acc_sh.at[pl.ds(base, STRIPE)], out_hbm.at[c, pl.ds(base, STRIPE)])


@functools.cache
def _sc_kernels():
    mesh = plsc.VectorSubcoreMesh(
        core_axis_name="c", subcore_axis_name="s",
        num_cores=NC, num_subcores=NS,
    )
    deg = functools.partial(
        pl.kernel,
        out_type=jax.ShapeDtypeStruct((NC, N_PAD), jnp.float32),
        mesh=mesh,
        scratch_types=[
            pltpu.VMEM((R, 128), jnp.int32),
            pltpu.VMEM((128,), jnp.float32),
            pltpu.VMEM((STRIPE,), jnp.float32),
            pltpu.VMEM_SHARED((N_PAD,), jnp.float32),
        ],
    )(_deg_body)
    scat = functools.partial(
        pl.kernel,
        out_type=jax.ShapeDtypeStruct((NC, N_PAD, DH), jnp.float32),
        mesh=mesh,
        scratch_types=[
            pltpu.VMEM((RPT, 128), jnp.int32),
            pltpu.VMEM((RPT, 128), jnp.int32),
            pltpu.VMEM((2, 128, DH), jnp.float32),
            pltpu.VMEM_SHARED((N_PAD, DH), jnp.float32),
            pltpu.SemaphoreType.DMA,
            pltpu.SemaphoreType.DMA,
        ],
        compiler_params=pltpu.CompilerParams(use_tc_tiling_on_sc=False),
    )(_scatter_body)
    return deg, scat


def _enc_body(x_ref, wen_ref, ben_ref, ae_ref, degp_ref, gc0w_ref,
              m2_ref, dinv_ref):
    deg = 1.0 + degp_ref[:, 0:1] + degp_ref[:, 1:2]
    dinv = lax.rsqrt(deg)
    dinv_ref[...] = dinv
    fc = lax.dot_general(x_ref[...], wen_ref[...], (((1,), (1,)), ((), ())),
                         preferred_element_type=jnp.float32) + ben_ref[...]
    ss = jnp.sum(fc * fc, axis=1, keepdims=True)
    nrm = jnp.maximum(jnp.sqrt(ss), 1e-12)
    en = ae_ref[...] * fc / nrm
    m0 = lax.dot_general(en, gc0w_ref[...], (((1,), (1,)), ((), ())),
                         preferred_element_type=jnp.float32)
    m2 = dinv * m0
    m2_ref[0] = m2[:, :DH]
    m2_ref[1] = m2[:, DH:]


_enc_call = pl.pallas_call(
    _enc_body,
    out_shape=(
        jax.ShapeDtypeStruct((NC, N, DH), jnp.float32),
        jax.ShapeDtypeStruct((N, 1), jnp.float32),
    ),
)


def _mid_body(a_ref, dinv_ref, bias_ref, g_ref, b_ref, w_ref,
              out_ref, *, use_bn):
    dinv = dinv_ref[...]

    def _half(hh, c):
        h = dinv * hh + bias_ref[:, c * DH:(c + 1) * DH]
        if use_bn:
            mu = jnp.mean(h, axis=0, keepdims=True)
            dd = h - mu
            var = jnp.mean(dd * dd, axis=0, keepdims=True)
            h = (g_ref[:, c * DH:(c + 1) * DH] * dd * lax.rsqrt(var + 1e-5)
                 + b_ref[:, c * DH:(c + 1) * DH])
        return h * jax.nn.sigmoid(h)

    h0 = _half(a_ref[0], 0)
    h1 = _half(a_ref[1], 1)
    m = (lax.dot_general(h0, w_ref[:, :DH], (((1,), (1,)), ((), ())),
                         preferred_element_type=jnp.float32)
         + lax.dot_general(h1, w_ref[:, DH:], (((1,), (1,)), ((), ())),
                           preferred_element_type=jnp.float32))
    m2 = dinv * m
    out_ref[0] = m2[:, :DH]
    out_ref[1] = m2[:, DH:]


_mid_call_bn = pl.pallas_call(
    functools.partial(_mid_body, use_bn=True),
    out_shape=jax.ShapeDtypeStruct((NC, N, DH), jnp.float32),
)
_mid_call = pl.pallas_call(
    functools.partial(_mid_body, use_bn=False),
    out_shape=jax.ShapeDtypeStruct((NC, N, DH), jnp.float32),
)


def _fin_body(a_ref, dinv_ref, bias_ref, bid_ref, mf_ref,
              fcm0w_ref, fcm0b_ref, fcm1w_ref, fcm1b_ref, bnmg_ref, bnmb_ref,
              fc0w_ref, fc0b_ref, fc1w_ref, fc1b_ref, fc2w_ref, fc2b_ref,
              out_ref):
    dinv = dinv_ref[...]
    bids = bid_ref[...]

    def _pool(c):
        h = dinv * a_ref[c] + bias_ref[:, c * DH:(c + 1) * DH]
        h = h * jax.nn.sigmoid(h)
        hg = jnp.zeros((B, DH), jnp.float32)
        CH = 2000
        for c0 in range(0, N, CH):
            blk = h[c0:c0 + CH]
            ids = bids[c0:c0 + CH]
            iot = lax.broadcasted_iota(jnp.int32, (CH, B), 1)
            oh = (ids == iot).astype(jnp.float32)
            hg = hg + lax.dot_general(oh, blk, (((0,), (0,)), ((), ())),
                                      preferred_element_type=jnp.float32)
        return hg

    hg0 = _pool(0)
    hg1 = _pool(1)

    hm = lax.dot_general(mf_ref[...], fcm0w_ref[...], (((1,), (1,)), ((), ())),
                         preferred_element_type=jnp.float32) + fcm0b_ref[...]
    mu = jnp.mean(hm, axis=0, keepdims=True)
    dd = hm - mu
    var = jnp.mean(dd * dd, axis=0, keepdims=True)
    hm = bnmg_ref[...] * dd * lax.rsqrt(var + 1e-5) + bnmb_ref[...]
    hm = hm * jax.nn.sigmoid(hm)
    hm = lax.dot_general(hm, fcm1w_ref[...], (((1,), (1,)), ((), ())),
                         preferred_element_type=jnp.float32) + fcm1b_ref[...]
    hm = hm * jax.nn.sigmoid(hm)

    z = (lax.dot_general(hg0, fc0w_ref[:, :DH], (((1,), (1,)), ((), ())),
                         preferred_element_type=jnp.float32)
         + lax.dot_general(hg1, fc0w_ref[:, DH:D], (((1,), (1,)), ((), ())),
                           preferred_element_type=jnp.float32)
         + lax.dot_general(hm, fc0w_ref[:, D:], (((1,), (1,)), ((), ())),
                           preferred_element_type=jnp.float32)) + fc0b_ref[...]
    z = z * jax.nn.sigmoid(z)
    z = lax.dot_general(z, fc1w_ref[...], (((1,), (1,)), ((), ())),
                        preferred_element_type=jnp.float32) + fc1b_ref[...]
    z = z * jax.nn.sigmoid(z)
    out_ref[...] = jnp.sum(z * fc2w_ref[...], axis=1, keepdims=True) + fc2b_ref[0, 0]


_fin_call = pl.pallas_call(
    _fin_body,
    out_shape=jax.ShapeDtypeStruct((B, 1), jnp.float32),
)


def kernel(x, edge_index, atom_ens, batch, mol_feats, W_en, b_en,
           gc0_W, gc0_b, gc1_W, gc1_b, gc2_W, gc2_b, bn_gc_g, bn_gc_b,
           fcm0_W, fcm0_b, fcm1_W, fcm1_b, bn_m_g, bn_m_b,
           fc0_W, fc0_b, fc1_W, fc1_b, fc2_W, fc2_b):
    src = edge_index[0]
    dst = edge_index[1]
    pad = E_PAD - E
    srcp = jnp.concatenate([src, jnp.zeros((pad,), jnp.int32)]).reshape(ROWS, 128)
    dstp = jnp.concatenate([dst, jnp.full((pad,), N, jnp.int32)]).reshape(ROWS, 128)

    deg_kernel, scatter_kernel = _sc_kernels()
    degp = deg_kernel(dstp)
    degp_t = degp[:, :N].T

    row = lambda v: v.reshape(1, -1)
    m2_0, dinv = _enc_call(x, W_en, row(b_en), atom_ens.reshape(-1, 1),
                           degp_t, gc0_W)

    def _agg(m2):
        m2p = jnp.pad(m2, ((0, 0), (0, N_PAD - N), (0, 0)))
        acc = scatter_kernel(m2p, srcp, dstp)
        return acc[:, :N]

    acc0 = _agg(m2_0)
    m2_1 = _mid_call_bn(acc0, dinv, row(gc0_b), row(bn_gc_g),
                        row(bn_gc_b), gc1_W)
    acc1 = _agg(m2_1)
    zz = row(jnp.zeros_like(gc1_b))
    m2_2 = _mid_call(acc1, dinv, row(gc1_b), zz, zz, gc2_W)
    acc2 = _agg(m2_2)

    out = _fin_call(acc2, dinv, row(gc2_b), batch.reshape(-1, 1),
                    mol_feats, fcm0_W, row(fcm0_b), fcm1_W, row(fcm1_b),
                    row(bn_m_g), row(bn_m_b), fc0_W, row(fc0_b),
                    fc1_W, row(fc1_b), fc2_W, row(fc2_b))
    return out
